# baseline (speedup 1.0000x reference)
import jax
import jax.numpy as jnp
from jax.experimental import pallas as pl
from jax.experimental.pallas import tpu as pltpu

_NUM_CLASSES = 90
_TOP_K = 200
_CONF = 0.05
_NMS_T = 0.5
_CROP = 300.0
_NEG = -1e9
_LANES = 128


def _score_cls_kernel(conf_ref, score_ref, cls_ref):
    c = conf_ref[...]
    m = jnp.max(c, axis=1, keepdims=True)
    iota = jax.lax.broadcasted_iota(jnp.int32, c.shape, 1)
    idx = jnp.min(jnp.where(c == m, iota, 2147483647), axis=1, keepdims=True)
    score_ref[...] = m
    cls_ref[...] = idx.astype(jnp.float32)


_SELR = 2


def _nms_kernel(score_ref, cls_ref, reg_ref, anch_ref, out_ref, clsout_ref,
                bx1_ref, by1_ref, bx2_ref, by2_ref,
                ox1_ref, oy1_ref, ox2_ref, oy2_ref, area_ref,
                s_ref, rowmax_ref,
                selx1_ref, sely1_ref, selx2_ref, sely2_ref, selar_ref):
    R = score_ref.shape[1]
    scores_raw = score_ref[0]
    clsf = cls_ref[0]
    ay1 = anch_ref[0]
    ax1 = anch_ref[1]
    ay2 = anch_ref[2]
    ax2 = anch_ref[3]
    dy = reg_ref[0, 0]
    dx = reg_ref[0, 1]
    dh = reg_ref[0, 2]
    dw = reg_ref[0, 3]
    yc_a = (ay1 + ay2) / 2.0
    xc_a = (ax1 + ax2) / 2.0
    ha = ay2 - ay1
    wa = ax2 - ax1
    w = jnp.exp(dw) * wa
    h = jnp.exp(dh) * ha
    yc = dy * ha + yc_a
    xc = dx * wa + xc_a
    bx1 = jnp.clip(xc - w / 2.0, 0.0, _CROP) / _CROP
    by1 = jnp.clip(yc - h / 2.0, 0.0, _CROP) / _CROP
    bx2 = jnp.clip(xc + w / 2.0, 0.0, _CROP) / _CROP
    by2 = jnp.clip(yc + h / 2.0, 0.0, _CROP) / _CROP
    off = clsf * 2.0
    ox1 = bx1 + off
    oy1 = by1 + off
    ox2 = bx2 + off
    oy2 = by2 + off
    area = jnp.clip(ox2 - ox1, 0.0, None) * jnp.clip(oy2 - oy1, 0.0, None)
    bx1_ref[...] = bx1
    by1_ref[...] = by1
    bx2_ref[...] = bx2
    by2_ref[...] = by2
    ox1_ref[...] = ox1
    oy1_ref[...] = oy1
    ox2_ref[...] = ox2
    oy2_ref[...] = oy2
    area_ref[...] = area

    lane1 = jax.lax.broadcasted_iota(jnp.int32, (1, _LANES), 1)
    riota = jax.lax.broadcasted_iota(jnp.int32, (R, 1), 0)
    s0 = jnp.where(scores_raw > _CONF, scores_raw, _NEG)
    s_ref[...] = s0
    rowmax_ref[...] = jnp.max(s0, axis=1, keepdims=True)
    selx1_ref[...] = jnp.zeros((_SELR, _LANES), jnp.float32)
    sely1_ref[...] = jnp.zeros((_SELR, _LANES), jnp.float32)
    selx2_ref[...] = jnp.zeros((_SELR, _LANES), jnp.float32)
    sely2_ref[...] = jnp.zeros((_SELR, _LANES), jnp.float32)
    selar_ref[...] = jnp.zeros((_SELR, _LANES), jnp.float32)
    out_ref[...] = jnp.zeros((1, _TOP_K, _LANES), jnp.float32)
    clsout_ref[...] = jnp.full((1, _TOP_K, _LANES), -1.0, jnp.float32)

    def gather(ref, ri, lmask):
        row = ref[pl.ds(ri, 1), :]
        return jnp.sum(jnp.where(lmask, row, 0.0))

    def cond(state):
        t, alive = state
        return (t < _TOP_K) & alive

    def body(state):
        t, _ = state
        rm = rowmax_ref[...]
        m = jnp.max(rm)
        alive = m > _NEG / 2.0

        def do_pop(t):
            rid = jnp.min(jnp.where(rm == m, riota, 2147483647))
            srow = s_ref[pl.ds(rid, 1), :]
            li = jnp.min(jnp.where(srow == m, lane1, 2147483647))
            lmask = lane1 == li
            nrow = jnp.where(lmask, _NEG, srow)
            s_ref[pl.ds(rid, 1), :] = nrow
            rowmax_ref[pl.ds(rid, 1), :] = jnp.max(nrow, axis=1, keepdims=True)
            cx1 = gather(ox1_ref, rid, lmask)
            cy1 = gather(oy1_ref, rid, lmask)
            cx2 = gather(ox2_ref, rid, lmask)
            cy2 = gather(oy2_ref, rid, lmask)
            car = gather(area_ref, rid, lmask)
            xx1 = jnp.maximum(cx1, selx1_ref[...])
            yy1 = jnp.maximum(cy1, sely1_ref[...])
            xx2 = jnp.minimum(cx2, selx2_ref[...])
            yy2 = jnp.minimum(cy2, sely2_ref[...])
            inter = jnp.clip(xx2 - xx1, 0.0, None) * jnp.clip(yy2 - yy1, 0.0, None)
            iou = inter / (selar_ref[...] + car - inter + 1e-8)
            sup = jnp.max(jnp.where(iou > _NMS_T, 1, 0)) > 0

            def accept(t):
                rt = t // _LANES
                lt = t - rt * _LANES
                ltm = lane1 == lt
                selx1_ref[pl.ds(rt, 1), :] = jnp.where(
                    ltm, cx1, selx1_ref[pl.ds(rt, 1), :])
                sely1_ref[pl.ds(rt, 1), :] = jnp.where(
                    ltm, cy1, sely1_ref[pl.ds(rt, 1), :])
                selx2_ref[pl.ds(rt, 1), :] = jnp.where(
                    ltm, cx2, selx2_ref[pl.ds(rt, 1), :])
                sely2_ref[pl.ds(rt, 1), :] = jnp.where(
                    ltm, cy2, sely2_ref[pl.ds(rt, 1), :])
                selar_ref[pl.ds(rt, 1), :] = jnp.where(
                    ltm, car, selar_ref[pl.ds(rt, 1), :])
                dx1 = gather(bx1_ref, rid, lmask)
                dy1 = gather(by1_ref, rid, lmask)
                dx2 = gather(bx2_ref, rid, lmask)
                dy2 = gather(by2_ref, rid, lmask)
                sscore = jnp.sum(jnp.where(lmask, score_ref[0, pl.ds(rid, 1), :], 0.0))
                scls = jnp.sum(jnp.where(lmask, cls_ref[0, pl.ds(rid, 1), :], 0.0))
                row = jnp.where(lane1 == 0, dx1, 0.0)
                row = jnp.where(lane1 == 1, dy1, row)
                row = jnp.where(lane1 == 2, dx2, row)
                row = jnp.where(lane1 == 3, dy2, row)
                row = jnp.where(lane1 == 4, sscore, row)
                out_ref[0, pl.ds(t, 1), :] = row
                clsout_ref[0, pl.ds(t, 1), :] = jnp.broadcast_to(scls, (1, _LANES))
                return t + 1

            return jax.lax.cond(sup, lambda tt: tt, accept, t)

        t2 = jax.lax.cond(alive, do_pop, lambda tt: tt, t)
        return (t2, alive)

    jax.lax.while_loop(cond, body, (jnp.int32(0), jnp.bool_(True)))


@jax.jit
def kernel(confidences, regressions, anchors):
    B, N, C = confidences.shape
    R = (N + _LANES - 1) // _LANES
    NP = R * _LANES
    pad = NP - N
    conf_p = jnp.pad(confidences, ((0, 0), (0, pad), (0, 0)))
    reg_p = jnp.pad(regressions, ((0, 0), (0, pad), (0, 0)))
    anch_p = jnp.pad(anchors, ((0, pad), (0, 0)))

    rows_total = B * NP
    blk = 1024
    while rows_total % blk != 0:
        blk //= 2
    conf2 = conf_p.reshape(rows_total, C)
    sc, cl = pl.pallas_call(
        _score_cls_kernel,
        grid=(rows_total // blk,),
        in_specs=[pl.BlockSpec((blk, C), lambda i: (i, 0))],
        out_specs=[pl.BlockSpec((blk, 1), lambda i: (i, 0)),
                   pl.BlockSpec((blk, 1), lambda i: (i, 0))],
        out_shape=[jax.ShapeDtypeStruct((rows_total, 1), jnp.float32),
                   jax.ShapeDtypeStruct((rows_total, 1), jnp.float32)],
    )(conf2)
    scores = sc.reshape(B, R, _LANES)
    clsf = cl.reshape(B, R, _LANES)
    reg_t = reg_p.transpose(0, 2, 1).reshape(B, 4, R, _LANES)
    anch_t = anch_p.T.reshape(4, R, _LANES)

    out_p, clsout_p = pl.pallas_call(
        _nms_kernel,
        grid=(B,),
        in_specs=[
            pl.BlockSpec((1, R, _LANES), lambda b: (b, 0, 0)),
            pl.BlockSpec((1, R, _LANES), lambda b: (b, 0, 0)),
            pl.BlockSpec((1, 4, R, _LANES), lambda b: (b, 0, 0, 0)),
            pl.BlockSpec((4, R, _LANES), lambda b: (0, 0, 0)),
        ],
        out_specs=[pl.BlockSpec((1, _TOP_K, _LANES), lambda b: (b, 0, 0)),
                   pl.BlockSpec((1, _TOP_K, _LANES), lambda b: (b, 0, 0))],
        out_shape=[jax.ShapeDtypeStruct((B, _TOP_K, _LANES), jnp.float32),
                   jax.ShapeDtypeStruct((B, _TOP_K, _LANES), jnp.float32)],
        scratch_shapes=[pltpu.VMEM((R, _LANES), jnp.float32)] * 10
        + [pltpu.VMEM((R, 1), jnp.float32)]
        + [pltpu.VMEM((_SELR, _LANES), jnp.float32)] * 5,
    )(scores, clsf, reg_t, anch_t)
    out = out_p[:, :, :5]
    out_classes = clsout_p[:, :, 0].astype(jnp.int32)
    return out, out_classes


# speedup vs baseline: 1.2199x; 1.2199x over previous
import jax
import jax.numpy as jnp
from jax.experimental import pallas as pl
from jax.experimental.pallas import tpu as pltpu

_NUM_CLASSES = 90
_TOP_K = 200
_CONF = 0.05
_NMS_T = 0.5
_CROP = 300.0
_NEG = -1e9
_LANES = 128
_SEL = 256
_STAGE = 208


def _score_cls_kernel(conf_ref, score_ref, cls_ref):
    c = conf_ref[...]
    m = jnp.max(c, axis=1, keepdims=True)
    iota = jax.lax.broadcasted_iota(jnp.int32, c.shape, 1)
    idx = jnp.min(jnp.where(c == m, iota, 2147483647), axis=1, keepdims=True)
    score_ref[...] = m
    cls_ref[...] = idx.astype(jnp.float32)


def _nms_kernel(score_ref, cls_ref, reg_ref, anch_ref, out_ref,
                bx1_ref, by1_ref, bx2_ref, by2_ref,
                ox1_ref, oy1_ref, ox2_ref, oy2_ref, area_ref,
                s_ref, stage_ref,
                selx1_ref, sely1_ref, selx2_ref, sely2_ref, selar_ref):
    B = score_ref.shape[0]
    R = score_ref.shape[1]
    scores_raw = score_ref[...]
    clsf = cls_ref[...]
    ay1 = anch_ref[0]
    ax1 = anch_ref[1]
    ay2 = anch_ref[2]
    ax2 = anch_ref[3]
    dy = reg_ref[:, 0]
    dx = reg_ref[:, 1]
    dh = reg_ref[:, 2]
    dw = reg_ref[:, 3]
    yc_a = (ay1 + ay2) / 2.0
    xc_a = (ax1 + ax2) / 2.0
    ha = ay2 - ay1
    wa = ax2 - ax1
    w = jnp.exp(dw) * wa
    h = jnp.exp(dh) * ha
    yc = dy * ha + yc_a
    xc = dx * wa + xc_a
    bx1 = jnp.clip(xc - w / 2.0, 0.0, _CROP) / _CROP
    by1 = jnp.clip(yc - h / 2.0, 0.0, _CROP) / _CROP
    bx2 = jnp.clip(xc + w / 2.0, 0.0, _CROP) / _CROP
    by2 = jnp.clip(yc + h / 2.0, 0.0, _CROP) / _CROP
    off = clsf * 2.0
    ox1 = bx1 + off
    oy1 = by1 + off
    ox2 = bx2 + off
    oy2 = by2 + off
    area = jnp.clip(ox2 - ox1, 0.0, None) * jnp.clip(oy2 - oy1, 0.0, None)
    bx1_ref[...] = bx1
    by1_ref[...] = by1
    bx2_ref[...] = bx2
    by2_ref[...] = by2
    ox1_ref[...] = ox1
    oy1_ref[...] = oy1
    ox2_ref[...] = ox2
    oy2_ref[...] = oy2
    area_ref[...] = area

    lane1 = jax.lax.broadcasted_iota(jnp.int32, (1, _LANES), 1)
    iota_sel = jax.lax.broadcasted_iota(jnp.int32, (1, _SEL), 1)
    s0 = jnp.where(scores_raw > _CONF, scores_raw, _NEG)
    s_ref[...] = s0
    zsel = jnp.zeros((B, _SEL), jnp.float32)
    selx1_ref[...] = zsel
    sely1_ref[...] = zsel
    selx2_ref[...] = zsel
    sely2_ref[...] = zsel
    selar_ref[...] = zsel
    stage_ref[...] = jnp.broadcast_to(
        jnp.where(lane1 == 5, -1.0, 0.0)[None], (B, _STAGE, _LANES))

    rm0 = jnp.max(s0, axis=2)
    rm_init = [
        jnp.concatenate(
            [rm0[b:b + 1, :], jnp.full((1, _SEL - R), _NEG, jnp.float32)],
            axis=1)
        for b in range(B)
    ]

    def gather(ref, b, ri, lmask):
        row = ref[b, pl.ds(ri, 1), :]
        return jnp.sum(jnp.where(lmask, row, 0.0))

    def cond(state):
        ts, alives, _ = state
        go = False
        for b in range(B):
            go = go | (alives[b] & (ts[b] < _TOP_K))
        return go

    def body(state):
        ts, alives, rms = state
        new_ts = []
        new_alives = []
        new_rms = []
        for b in range(B):
            t = ts[b]
            rm = rms[b]
            m = jnp.max(rm)
            alive = alives[b] & (m > _NEG / 2.0)
            rid = jnp.min(jnp.where(rm == m, iota_sel, 2147483647))
            srow = s_ref[b, pl.ds(rid, 1), :]
            li = jnp.min(jnp.where(srow == m, lane1, 2147483647))
            lmask = lane1 == li
            nrow = jnp.where(lmask, _NEG, srow)
            s_ref[b, pl.ds(rid, 1), :] = nrow
            rm = jnp.where(iota_sel == rid, jnp.max(nrow), rm)
            cx1 = gather(ox1_ref, b, rid, lmask)
            cy1 = gather(oy1_ref, b, rid, lmask)
            cx2 = gather(ox2_ref, b, rid, lmask)
            cy2 = gather(oy2_ref, b, rid, lmask)
            car = gather(area_ref, b, rid, lmask)
            xx1 = jnp.maximum(cx1, selx1_ref[b:b + 1, :])
            yy1 = jnp.maximum(cy1, sely1_ref[b:b + 1, :])
            xx2 = jnp.minimum(cx2, selx2_ref[b:b + 1, :])
            yy2 = jnp.minimum(cy2, sely2_ref[b:b + 1, :])
            inter = (jnp.clip(xx2 - xx1, 0.0, None)
                     * jnp.clip(yy2 - yy1, 0.0, None))
            iou = inter / (selar_ref[b:b + 1, :] + car - inter + 1e-8)
            sup = jnp.max(jnp.where(iou > _NMS_T, 1, 0)) > 0
            accept = alive & jnp.logical_not(sup) & (t < _TOP_K)
            amask = (iota_sel == t) & accept
            selx1_ref[b:b + 1, :] = jnp.where(amask, cx1, selx1_ref[b:b + 1, :])
            sely1_ref[b:b + 1, :] = jnp.where(amask, cy1, sely1_ref[b:b + 1, :])
            selx2_ref[b:b + 1, :] = jnp.where(amask, cx2, selx2_ref[b:b + 1, :])
            sely2_ref[b:b + 1, :] = jnp.where(amask, cy2, sely2_ref[b:b + 1, :])
            selar_ref[b:b + 1, :] = jnp.where(amask, car, selar_ref[b:b + 1, :])
            dx1 = gather(bx1_ref, b, rid, lmask)
            dy1 = gather(by1_ref, b, rid, lmask)
            dx2 = gather(bx2_ref, b, rid, lmask)
            dy2 = gather(by2_ref, b, rid, lmask)
            sscore = jnp.sum(jnp.where(lmask, srow, 0.0))
            scls = gather(cls_ref, b, rid, lmask)
            row = jnp.where(lane1 == 0, dx1, 0.0)
            row = jnp.where(lane1 == 1, dy1, row)
            row = jnp.where(lane1 == 2, dx2, row)
            row = jnp.where(lane1 == 3, dy2, row)
            row = jnp.where(lane1 == 4, sscore, row)
            row = jnp.where(lane1 == 5, scls, row)
            t_eff = jnp.where(accept, t, _TOP_K)
            stage_ref[b, pl.ds(t_eff, 1), :] = row
            new_ts.append(t + accept.astype(jnp.int32))
            new_alives.append(alive)
            new_rms.append(rm)
        return (tuple(new_ts), tuple(new_alives), tuple(new_rms))

    jax.lax.while_loop(
        cond, body,
        (tuple(jnp.int32(0) for _ in range(B)),
         tuple(jnp.bool_(True) for _ in range(B)),
         tuple(rm_init)))

    out_ref[...] = stage_ref[:, 0:_TOP_K, :]


@jax.jit
def kernel(confidences, regressions, anchors):
    B, N, C = confidences.shape
    R = (N + _LANES - 1) // _LANES
    NP = R * _LANES
    pad = NP - N
    conf_p = jnp.pad(confidences, ((0, 0), (0, pad), (0, 0)))
    reg_p = jnp.pad(regressions, ((0, 0), (0, pad), (0, 0)))
    anch_p = jnp.pad(anchors, ((0, pad), (0, 0)))

    rows_total = B * NP
    blk = 1024
    while rows_total % blk != 0:
        blk //= 2
    conf2 = conf_p.reshape(rows_total, C)
    sc, cl = pl.pallas_call(
        _score_cls_kernel,
        grid=(rows_total // blk,),
        in_specs=[pl.BlockSpec((blk, C), lambda i: (i, 0))],
        out_specs=[pl.BlockSpec((blk, 1), lambda i: (i, 0)),
                   pl.BlockSpec((blk, 1), lambda i: (i, 0))],
        out_shape=[jax.ShapeDtypeStruct((rows_total, 1), jnp.float32),
                   jax.ShapeDtypeStruct((rows_total, 1), jnp.float32)],
    )(conf2)
    scores = sc.reshape(B, R, _LANES)
    clsf = cl.reshape(B, R, _LANES)
    reg_t = reg_p.transpose(0, 2, 1).reshape(B, 4, R, _LANES)
    anch_t = anch_p.T.reshape(4, R, _LANES)

    (out_p,) = pl.pallas_call(
        _nms_kernel,
        grid=(1,),
        in_specs=[
            pl.BlockSpec((B, R, _LANES), lambda i: (0, 0, 0)),
            pl.BlockSpec((B, R, _LANES), lambda i: (0, 0, 0)),
            pl.BlockSpec((B, 4, R, _LANES), lambda i: (0, 0, 0, 0)),
            pl.BlockSpec((4, R, _LANES), lambda i: (0, 0, 0)),
        ],
        out_specs=[pl.BlockSpec((B, _TOP_K, _LANES), lambda i: (0, 0, 0))],
        out_shape=[jax.ShapeDtypeStruct((B, _TOP_K, _LANES), jnp.float32)],
        scratch_shapes=[pltpu.VMEM((B, R, _LANES), jnp.float32)] * 10
        + [pltpu.VMEM((B, _STAGE, _LANES), jnp.float32)]
        + [pltpu.VMEM((B, _SEL), jnp.float32)] * 5,
    )(scores, clsf, reg_t, anch_t)
    out = out_p[:, :, 0:5]
    out_classes = out_p[:, :, 5].astype(jnp.int32)
    return out, out_classes


# speedup vs baseline: 1.2623x; 1.0347x over previous
import jax
import jax.numpy as jnp
from jax.experimental import pallas as pl
from jax.experimental.pallas import tpu as pltpu

_NUM_CLASSES = 90
_TOP_K = 200
_CONF = 0.05
_NMS_T = 0.5
_CROP = 300.0
_NEG = -1e9
_LANES = 128
_SEL = 256
_OUTR = 208


def _score_cls_kernel(conf_ref, score_ref, cls_ref):
    c = conf_ref[...]
    m = jnp.max(c, axis=1, keepdims=True)
    iota = jax.lax.broadcasted_iota(jnp.int32, c.shape, 1)
    idx = jnp.min(jnp.where(c == m, iota, 2147483647), axis=1, keepdims=True)
    score_ref[...] = m
    cls_ref[...] = idx.astype(jnp.float32)


def _nms_kernel(score_ref, cls_ref, reg_ref, anch_ref, out_ref, *scratch):
    B = score_ref.shape[0]
    R = score_ref.shape[1]
    bx1_ref, by1_ref, bx2_ref, by2_ref = scratch[0:4]
    s_refs = scratch[4:4 + B]
    sel_refs = scratch[4 + B:]

    scores_raw = score_ref[...]
    clsf = cls_ref[...]
    ay1 = anch_ref[0]
    ax1 = anch_ref[1]
    ay2 = anch_ref[2]
    ax2 = anch_ref[3]
    dy = reg_ref[:, 0]
    dx = reg_ref[:, 1]
    dh = reg_ref[:, 2]
    dw = reg_ref[:, 3]
    yc_a = (ay1 + ay2) / 2.0
    xc_a = (ax1 + ax2) / 2.0
    ha = ay2 - ay1
    wa = ax2 - ax1
    w = jnp.exp(dw) * wa
    h = jnp.exp(dh) * ha
    yc = dy * ha + yc_a
    xc = dx * wa + xc_a
    bx1_ref[...] = jnp.clip(xc - w / 2.0, 0.0, _CROP) / _CROP
    by1_ref[...] = jnp.clip(yc - h / 2.0, 0.0, _CROP) / _CROP
    bx2_ref[...] = jnp.clip(xc + w / 2.0, 0.0, _CROP) / _CROP
    by2_ref[...] = jnp.clip(yc + h / 2.0, 0.0, _CROP) / _CROP

    lane1 = jax.lax.broadcasted_iota(jnp.int32, (1, _LANES), 1)
    iota_sel = jax.lax.broadcasted_iota(jnp.int32, (1, _SEL), 1)
    s0 = jnp.where(scores_raw > _CONF, scores_raw, _NEG)
    zsel = jnp.zeros((1, _SEL), jnp.float32)
    for b in range(B):
        s_refs[b][...] = s0[b]
        for f in range(5):
            sel_refs[5 * b + f][...] = zsel
    out_ref[...] = jnp.broadcast_to(
        jnp.where(lane1 == 5, -1.0, 0.0)[None], (B, _OUTR, _LANES))

    rm0 = jnp.max(s0, axis=2)
    rm_init = [
        jnp.concatenate(
            [rm0[b:b + 1, :], jnp.full((1, _SEL - R), _NEG, jnp.float32)],
            axis=1)
        for b in range(B)
    ]

    def rowsel(ref3, b, ri, lmask):
        row = ref3[b, pl.ds(ri, 1), :]
        return jnp.sum(jnp.where(lmask, row, 0.0))

    def cond(state):
        ts, _ = state
        go = False
        for b in range(B):
            go = go | (ts[b] < _TOP_K)
        return go

    def body(state):
        ts, rms = state
        new_ts = []
        new_rms = []
        for b in range(B):
            t = ts[b]
            rm = rms[b]
            m = jnp.max(rm)
            alive_v = m > _NEG / 2.0
            rid_v = jnp.min(jnp.where(rm == m, iota_sel, 2147483647))
            rid = jax.lax.convert_element_type(rid_v, jnp.int32)
            srow = s_refs[b][pl.ds(rid, 1), :]
            li_v = jnp.min(jnp.where(srow == m, lane1, 2147483647))
            lmask = lane1 == li_v
            nrow = jnp.where(lmask, _NEG, srow)
            s_refs[b][pl.ds(rid, 1), :] = nrow
            rm = jnp.where(iota_sel == rid_v, jnp.max(nrow), rm)
            cbx1 = rowsel(bx1_ref, b, rid, lmask)
            cby1 = rowsel(by1_ref, b, rid, lmask)
            cbx2 = rowsel(bx2_ref, b, rid, lmask)
            cby2 = rowsel(by2_ref, b, rid, lmask)
            ccls = rowsel(cls_ref, b, rid, lmask)
            coff = ccls * 2.0
            cx1 = cbx1 + coff
            cy1 = cby1 + coff
            cx2 = cbx2 + coff
            cy2 = cby2 + coff
            car = (jnp.clip(cx2 - cx1, 0.0, None)
                   * jnp.clip(cy2 - cy1, 0.0, None))
            sx1 = sel_refs[5 * b + 0][...]
            sy1 = sel_refs[5 * b + 1][...]
            sx2 = sel_refs[5 * b + 2][...]
            sy2 = sel_refs[5 * b + 3][...]
            sar = sel_refs[5 * b + 4][...]
            xx1 = jnp.maximum(cx1, sx1)
            yy1 = jnp.maximum(cy1, sy1)
            xx2 = jnp.minimum(cx2, sx2)
            yy2 = jnp.minimum(cy2, sy2)
            inter = (jnp.clip(xx2 - xx1, 0.0, None)
                     * jnp.clip(yy2 - yy1, 0.0, None))
            iou = inter / (sar + car - inter + 1e-8)
            sup = jnp.max(jnp.where(iou > _NMS_T, 1, 0)) > 0
            accept_v = alive_v & jnp.logical_not(sup) & (t < _TOP_K)
            flags = (jnp.where(accept_v, 2, 0)
                     + jnp.where(alive_v, 1, 0))
            fs = jax.lax.convert_element_type(flags, jnp.int32)
            accept_s = fs == 3
            alive_s = fs >= 1
            amask = (iota_sel == t) & accept_v
            sel_refs[5 * b + 0][...] = jnp.where(amask, cx1, sx1)
            sel_refs[5 * b + 1][...] = jnp.where(amask, cy1, sy1)
            sel_refs[5 * b + 2][...] = jnp.where(amask, cx2, sx2)
            sel_refs[5 * b + 3][...] = jnp.where(amask, cy2, sy2)
            sel_refs[5 * b + 4][...] = jnp.where(amask, car, sar)
            row = jnp.where(lane1 == 0, cbx1, 0.0)
            row = jnp.where(lane1 == 1, cby1, row)
            row = jnp.where(lane1 == 2, cbx2, row)
            row = jnp.where(lane1 == 3, cby2, row)
            row = jnp.where(lane1 == 4, m, row)
            row = jnp.where(lane1 == 5, ccls, row)
            t_eff = jnp.where(accept_s, t, _TOP_K)
            out_ref[b, pl.ds(t_eff, 1), :] = row
            new_ts.append(jnp.where(alive_s, t + accept_s.astype(jnp.int32),
                                    _TOP_K))
            new_rms.append(rm)
        return (tuple(new_ts), tuple(new_rms))

    jax.lax.while_loop(
        cond, body,
        (tuple(jnp.int32(0) for _ in range(B)), tuple(rm_init)))


@jax.jit
def kernel(confidences, regressions, anchors):
    B, N, C = confidences.shape
    R = (N + _LANES - 1) // _LANES
    NP = R * _LANES
    pad = NP - N

    rows_total = B * N
    blk = 1024
    while rows_total % blk != 0 or blk % 8 != 0:
        blk -= 8
    conf2 = confidences.reshape(rows_total, C)
    sc, cl = pl.pallas_call(
        _score_cls_kernel,
        grid=(rows_total // blk,),
        in_specs=[pl.BlockSpec((blk, C), lambda i: (i, 0))],
        out_specs=[pl.BlockSpec((blk, 1), lambda i: (i, 0)),
                   pl.BlockSpec((blk, 1), lambda i: (i, 0))],
        out_shape=[jax.ShapeDtypeStruct((rows_total, 1), jnp.float32),
                   jax.ShapeDtypeStruct((rows_total, 1), jnp.float32)],
    )(conf2)
    scores = jnp.pad(sc.reshape(B, N), ((0, 0), (0, pad))).reshape(B, R, _LANES)
    clsf = jnp.pad(cl.reshape(B, N), ((0, 0), (0, pad))).reshape(B, R, _LANES)
    reg_p = jnp.pad(regressions, ((0, 0), (0, pad), (0, 0)))
    anch_p = jnp.pad(anchors, ((0, pad), (0, 0)))
    reg_t = reg_p.transpose(0, 2, 1).reshape(B, 4, R, _LANES)
    anch_t = anch_p.T.reshape(4, R, _LANES)

    (out_p,) = pl.pallas_call(
        _nms_kernel,
        grid=(1,),
        in_specs=[
            pl.BlockSpec((B, R, _LANES), lambda i: (0, 0, 0)),
            pl.BlockSpec((B, R, _LANES), lambda i: (0, 0, 0)),
            pl.BlockSpec((B, 4, R, _LANES), lambda i: (0, 0, 0, 0)),
            pl.BlockSpec((4, R, _LANES), lambda i: (0, 0, 0)),
        ],
        out_specs=[pl.BlockSpec((B, _OUTR, _LANES), lambda i: (0, 0, 0))],
        out_shape=[jax.ShapeDtypeStruct((B, _OUTR, _LANES), jnp.float32)],
        scratch_shapes=[pltpu.VMEM((B, R, _LANES), jnp.float32)] * 4
        + [pltpu.VMEM((R, _LANES), jnp.float32)] * B
        + [pltpu.VMEM((1, _SEL), jnp.float32)] * (5 * B),
    )(scores, clsf, reg_t, anch_t)
    out = out_p[:, :_TOP_K, 0:5]
    out_classes = out_p[:, :_TOP_K, 5].astype(jnp.int32)
    return out, out_classes


# speedup vs baseline: 3.0713x; 2.4332x over previous
import jax
import jax.numpy as jnp
from jax.experimental import pallas as pl
from jax.experimental.pallas import tpu as pltpu

_NUM_CLASSES = 90
_TOP_K = 200
_CONF = 0.05
_NMS_T = 0.5
_CROP = 300.0
_NEG = -1e9
_LANES = 128
_SEL = 256
_OUTR = 208


def _score_cls_kernel(conf_ref, score_ref, cls_ref):
    c = conf_ref[...]
    m = jnp.max(c, axis=1, keepdims=True)
    iota = jax.lax.broadcasted_iota(jnp.int32, c.shape, 1)
    idx = jnp.min(jnp.where(c == m, iota, 2147483647), axis=1, keepdims=True)
    score_ref[...] = m
    cls_ref[...] = idx.astype(jnp.float32)


def _nms_kernel(score_ref, cls_ref, reg_ref, anch_ref, out_ref, *scratch):
    B = score_ref.shape[0]
    R = score_ref.shape[1]
    bx1_ref, by1_ref, bx2_ref, by2_ref = scratch[0:4]
    s_refs = scratch[4:4 + B]
    sel_refs = scratch[4 + B:]

    scores_raw = score_ref[...]
    clsf = cls_ref[...]
    ay1 = anch_ref[0]
    ax1 = anch_ref[1]
    ay2 = anch_ref[2]
    ax2 = anch_ref[3]
    dy = reg_ref[:, 0]
    dx = reg_ref[:, 1]
    dh = reg_ref[:, 2]
    dw = reg_ref[:, 3]
    yc_a = (ay1 + ay2) / 2.0
    xc_a = (ax1 + ax2) / 2.0
    ha = ay2 - ay1
    wa = ax2 - ax1
    w = jnp.exp(dw) * wa
    h = jnp.exp(dh) * ha
    yc = dy * ha + yc_a
    xc = dx * wa + xc_a
    bx1_ref[...] = jnp.clip(xc - w / 2.0, 0.0, _CROP) / _CROP
    by1_ref[...] = jnp.clip(yc - h / 2.0, 0.0, _CROP) / _CROP
    bx2_ref[...] = jnp.clip(xc + w / 2.0, 0.0, _CROP) / _CROP
    by2_ref[...] = jnp.clip(yc + h / 2.0, 0.0, _CROP) / _CROP

    lane1 = jax.lax.broadcasted_iota(jnp.int32, (1, _LANES), 1)
    iota_sel = jax.lax.broadcasted_iota(jnp.int32, (1, _SEL), 1)
    s0 = jnp.where(scores_raw > _CONF, scores_raw, _NEG)
    zsel = jnp.zeros((1, _SEL), jnp.float32)
    for b in range(B):
        s_refs[b][...] = s0[b]
        for f in range(5):
            sel_refs[5 * b + f][...] = zsel
    out_ref[...] = jnp.broadcast_to(
        jnp.where(lane1 == 5, -1.0, 0.0)[None], (B, _OUTR, _LANES))

    rm0 = jnp.max(s0, axis=2)
    rm_init = [
        jnp.concatenate(
            [rm0[b:b + 1, :], jnp.full((1, _SEL - R), _NEG, jnp.float32)],
            axis=1)
        for b in range(B)
    ]

    def rowsel(ref3, b, ri, lmask):
        row = ref3[b, pl.ds(ri, 1), :]
        return jnp.sum(jnp.where(lmask, row, 0.0))

    def cond(state):
        ts, _ = state
        go = False
        for b in range(B):
            go = go | (ts[b] < _TOP_K)
        return go

    def body(state):
        ts, rms = state
        bs = range(B)
        ms = [jnp.max(rms[b]) for b in bs]
        alive_vs = [ms[b] > _NEG / 2.0 for b in bs]
        rid_vs = [jnp.min(jnp.where(rms[b] == ms[b], iota_sel, 2147483647))
                  for b in bs]
        rids = [jax.lax.convert_element_type(rid_vs[b], jnp.int32) for b in bs]
        srows = [s_refs[b][pl.ds(rids[b], 1), :] for b in bs]
        li_vs = [jnp.min(jnp.where(srows[b] == ms[b], lane1, 2147483647))
                 for b in bs]
        lmasks = [lane1 == li_vs[b] for b in bs]
        nrows = [jnp.where(lmasks[b], _NEG, srows[b]) for b in bs]
        for b in bs:
            s_refs[b][pl.ds(rids[b], 1), :] = nrows[b]
        nrmaxs = [jnp.max(nrows[b]) for b in bs]
        new_rms = [jnp.where(iota_sel == rid_vs[b], nrmaxs[b], rms[b])
                   for b in bs]
        cbx1s = [rowsel(bx1_ref, b, rids[b], lmasks[b]) for b in bs]
        cby1s = [rowsel(by1_ref, b, rids[b], lmasks[b]) for b in bs]
        cbx2s = [rowsel(bx2_ref, b, rids[b], lmasks[b]) for b in bs]
        cby2s = [rowsel(by2_ref, b, rids[b], lmasks[b]) for b in bs]
        cclss = [rowsel(cls_ref, b, rids[b], lmasks[b]) for b in bs]
        coffs = [cclss[b] * 2.0 for b in bs]
        cx1s = [cbx1s[b] + coffs[b] for b in bs]
        cy1s = [cby1s[b] + coffs[b] for b in bs]
        cx2s = [cbx2s[b] + coffs[b] for b in bs]
        cy2s = [cby2s[b] + coffs[b] for b in bs]
        cars = [(jnp.clip(cx2s[b] - cx1s[b], 0.0, None)
                 * jnp.clip(cy2s[b] - cy1s[b], 0.0, None)) for b in bs]
        sxs = [[sel_refs[5 * b + f][...] for f in range(5)] for b in bs]
        sups = []
        for b in bs:
            sx1, sy1, sx2, sy2, sar = sxs[b]
            xx1 = jnp.maximum(cx1s[b], sx1)
            yy1 = jnp.maximum(cy1s[b], sy1)
            xx2 = jnp.minimum(cx2s[b], sx2)
            yy2 = jnp.minimum(cy2s[b], sy2)
            inter = (jnp.clip(xx2 - xx1, 0.0, None)
                     * jnp.clip(yy2 - yy1, 0.0, None))
            iou = inter / (sar + cars[b] - inter + 1e-8)
            sups.append(jnp.max(jnp.where(iou > _NMS_T, 1, 0)) > 0)
        accept_vs = [alive_vs[b] & jnp.logical_not(sups[b]) & (ts[b] < _TOP_K)
                     for b in bs]
        flagss = [(jnp.where(accept_vs[b], 2, 0)
                   + jnp.where(alive_vs[b], 1, 0)) for b in bs]
        fss = [jax.lax.convert_element_type(flagss[b], jnp.int32) for b in bs]
        accept_ss = [fss[b] == 3 for b in bs]
        alive_ss = [fss[b] >= 1 for b in bs]
        new_ts = []
        for b in bs:
            t = ts[b]
            amask = (iota_sel == t) & accept_vs[b]
            cvals = [cx1s[b], cy1s[b], cx2s[b], cy2s[b], cars[b]]
            for f in range(5):
                sel_refs[5 * b + f][...] = jnp.where(amask, cvals[f],
                                                     sxs[b][f])
            row = jnp.where(lane1 == 0, cbx1s[b], 0.0)
            row = jnp.where(lane1 == 1, cby1s[b], row)
            row = jnp.where(lane1 == 2, cbx2s[b], row)
            row = jnp.where(lane1 == 3, cby2s[b], row)
            row = jnp.where(lane1 == 4, ms[b], row)
            row = jnp.where(lane1 == 5, cclss[b], row)
            t_eff = jnp.where(accept_ss[b], t, _TOP_K)
            out_ref[b, pl.ds(t_eff, 1), :] = row
            new_ts.append(jnp.where(alive_ss[b],
                                    t + accept_ss[b].astype(jnp.int32),
                                    _TOP_K))
        return (tuple(new_ts), tuple(new_rms))

    jax.lax.while_loop(
        cond, body,
        (tuple(jnp.int32(0) for _ in range(B)), tuple(rm_init)))


@jax.jit
def kernel(confidences, regressions, anchors):
    B, N, C = confidences.shape
    R = (N + _LANES - 1) // _LANES
    NP = R * _LANES
    pad = NP - N

    rows_total = B * N
    blk = 1024
    while rows_total % blk != 0 or blk % 8 != 0:
        blk -= 8
    conf2 = confidences.reshape(rows_total, C)
    sc, cl = pl.pallas_call(
        _score_cls_kernel,
        grid=(rows_total // blk,),
        in_specs=[pl.BlockSpec((blk, C), lambda i: (i, 0))],
        out_specs=[pl.BlockSpec((blk, 1), lambda i: (i, 0)),
                   pl.BlockSpec((blk, 1), lambda i: (i, 0))],
        out_shape=[jax.ShapeDtypeStruct((rows_total, 1), jnp.float32),
                   jax.ShapeDtypeStruct((rows_total, 1), jnp.float32)],
    )(conf2)
    scores = jnp.pad(sc.reshape(B, N), ((0, 0), (0, pad))).reshape(B, R, _LANES)
    clsf = jnp.pad(cl.reshape(B, N), ((0, 0), (0, pad))).reshape(B, R, _LANES)
    reg_p = jnp.pad(regressions, ((0, 0), (0, pad), (0, 0)))
    anch_p = jnp.pad(anchors, ((0, pad), (0, 0)))
    reg_t = reg_p.transpose(0, 2, 1).reshape(B, 4, R, _LANES)
    anch_t = anch_p.T.reshape(4, R, _LANES)

    (out_p,) = pl.pallas_call(
        _nms_kernel,
        grid=(1,),
        in_specs=[
            pl.BlockSpec((B, R, _LANES), lambda i: (0, 0, 0)),
            pl.BlockSpec((B, R, _LANES), lambda i: (0, 0, 0)),
            pl.BlockSpec((B, 4, R, _LANES), lambda i: (0, 0, 0, 0)),
            pl.BlockSpec((4, R, _LANES), lambda i: (0, 0, 0)),
        ],
        out_specs=[pl.BlockSpec((B, _OUTR, _LANES), lambda i: (0, 0, 0))],
        out_shape=[jax.ShapeDtypeStruct((B, _OUTR, _LANES), jnp.float32)],
        scratch_shapes=[pltpu.VMEM((B, R, _LANES), jnp.float32)] * 4
        + [pltpu.VMEM((R, _LANES), jnp.float32)] * B
        + [pltpu.VMEM((1, _SEL), jnp.float32)] * (5 * B),
    )(scores, clsf, reg_t, anch_t)
    out = out_p[:, :_TOP_K, 0:5]
    out_classes = out_p[:, :_TOP_K, 5].astype(jnp.int32)
    return out, out_classes


# speedup vs baseline: 4.2426x; 1.3814x over previous
import jax
import jax.numpy as jnp
from jax.experimental import pallas as pl
from jax.experimental.pallas import tpu as pltpu

_NUM_CLASSES = 90
_TOP_K = 200
_CONF = 0.05
_NMS_T = 0.5
_CROP = 300.0
_NEG = -1e9
_LANES = 128
_SEL = 256
_OUTR = 208


def _score_cls_kernel(conf_ref, score_ref, cls_ref):
    c = conf_ref[0]
    m = jnp.max(c, axis=1, keepdims=True)
    iota = jax.lax.broadcasted_iota(jnp.int32, c.shape, 1)
    idx = jnp.min(jnp.where(c == m, iota, 2147483647), axis=1, keepdims=True)
    score_ref[0] = m
    cls_ref[0] = idx.astype(jnp.float32)


def _nms_kernel(score_ref, cls_ref, reg_ref, anch_ref, out_ref, *scratch):
    B = score_ref.shape[0]
    R = score_ref.shape[1]
    bx1_ref, by1_ref, bx2_ref, by2_ref = scratch[0:4]
    s_refs = scratch[4:4 + B]
    sel_refs = scratch[4 + B:]

    scores_raw = score_ref[...]
    clsf = cls_ref[...]
    ay1 = anch_ref[0]
    ax1 = anch_ref[1]
    ay2 = anch_ref[2]
    ax2 = anch_ref[3]
    dy = reg_ref[:, 0]
    dx = reg_ref[:, 1]
    dh = reg_ref[:, 2]
    dw = reg_ref[:, 3]
    yc_a = (ay1 + ay2) / 2.0
    xc_a = (ax1 + ax2) / 2.0
    ha = ay2 - ay1
    wa = ax2 - ax1
    w = jnp.exp(dw) * wa
    h = jnp.exp(dh) * ha
    yc = dy * ha + yc_a
    xc = dx * wa + xc_a
    bx1_ref[...] = jnp.clip(xc - w / 2.0, 0.0, _CROP) / _CROP
    by1_ref[...] = jnp.clip(yc - h / 2.0, 0.0, _CROP) / _CROP
    bx2_ref[...] = jnp.clip(xc + w / 2.0, 0.0, _CROP) / _CROP
    by2_ref[...] = jnp.clip(yc + h / 2.0, 0.0, _CROP) / _CROP

    lane1 = jax.lax.broadcasted_iota(jnp.int32, (1, _LANES), 1)
    iota_sel = jax.lax.broadcasted_iota(jnp.int32, (1, _SEL), 1)
    s0 = jnp.where(scores_raw > _CONF, scores_raw, _NEG)
    zsel = jnp.zeros((1, _SEL), jnp.float32)
    for b in range(B):
        s_refs[b][...] = s0[b]
        for f in range(5):
            sel_refs[5 * b + f][...] = zsel
    out_ref[...] = jnp.broadcast_to(
        jnp.where(lane1 == 5, -1.0, 0.0)[None], (B, _OUTR, _LANES))

    rm0 = jnp.max(s0, axis=2)
    rm_init = [
        jnp.concatenate(
            [rm0[b:b + 1, :], jnp.full((1, _SEL - R), _NEG, jnp.float32)],
            axis=1)
        for b in range(B)
    ]

    def rowsel(ref3, b, ri, lmask):
        row = ref3[b, pl.ds(ri, 1), :]
        return jnp.sum(jnp.where(lmask, row, 0.0))

    def cond(state):
        ts, _ = state
        go = False
        for b in range(B):
            go = go | (ts[b] < _TOP_K)
        return go

    def body(state):
        ts, rms = state
        bs = range(B)
        ms = [jnp.max(rms[b]) for b in bs]
        alive_vs = [ms[b] > _NEG / 2.0 for b in bs]
        rid_vs = [jnp.min(jnp.where(rms[b] == ms[b], iota_sel, 2147483647))
                  for b in bs]
        rids = [jax.lax.convert_element_type(rid_vs[b], jnp.int32) for b in bs]
        srows = [s_refs[b][pl.ds(rids[b], 1), :] for b in bs]
        li_vs = [jnp.min(jnp.where(srows[b] == ms[b], lane1, 2147483647))
                 for b in bs]
        lmasks = [lane1 == li_vs[b] for b in bs]
        nrows = [jnp.where(lmasks[b], _NEG, srows[b]) for b in bs]
        for b in bs:
            s_refs[b][pl.ds(rids[b], 1), :] = nrows[b]
        nrmaxs = [jnp.max(nrows[b]) for b in bs]
        new_rms = [jnp.where(iota_sel == rid_vs[b], nrmaxs[b], rms[b])
                   for b in bs]
        cbx1s = [rowsel(bx1_ref, b, rids[b], lmasks[b]) for b in bs]
        cby1s = [rowsel(by1_ref, b, rids[b], lmasks[b]) for b in bs]
        cbx2s = [rowsel(bx2_ref, b, rids[b], lmasks[b]) for b in bs]
        cby2s = [rowsel(by2_ref, b, rids[b], lmasks[b]) for b in bs]
        cclss = [rowsel(cls_ref, b, rids[b], lmasks[b]) for b in bs]
        coffs = [cclss[b] * 2.0 for b in bs]
        cx1s = [cbx1s[b] + coffs[b] for b in bs]
        cy1s = [cby1s[b] + coffs[b] for b in bs]
        cx2s = [cbx2s[b] + coffs[b] for b in bs]
        cy2s = [cby2s[b] + coffs[b] for b in bs]
        cars = [(jnp.clip(cx2s[b] - cx1s[b], 0.0, None)
                 * jnp.clip(cy2s[b] - cy1s[b], 0.0, None)) for b in bs]
        sxs = [[sel_refs[5 * b + f][...] for f in range(5)] for b in bs]
        sups = []
        for b in bs:
            sx1, sy1, sx2, sy2, sar = sxs[b]
            xx1 = jnp.maximum(cx1s[b], sx1)
            yy1 = jnp.maximum(cy1s[b], sy1)
            xx2 = jnp.minimum(cx2s[b], sx2)
            yy2 = jnp.minimum(cy2s[b], sy2)
            inter = (jnp.clip(xx2 - xx1, 0.0, None)
                     * jnp.clip(yy2 - yy1, 0.0, None))
            iou = inter / (sar + cars[b] - inter + 1e-8)
            sups.append(jnp.max(jnp.where(iou > _NMS_T, 1, 0)) > 0)
        accept_vs = [alive_vs[b] & jnp.logical_not(sups[b]) & (ts[b] < _TOP_K)
                     for b in bs]
        flagss = [(jnp.where(accept_vs[b], 2, 0)
                   + jnp.where(alive_vs[b], 1, 0)) for b in bs]
        fss = [jax.lax.convert_element_type(flagss[b], jnp.int32) for b in bs]
        accept_ss = [fss[b] == 3 for b in bs]
        alive_ss = [fss[b] >= 1 for b in bs]
        new_ts = []
        for b in bs:
            t = ts[b]
            amask = (iota_sel == t) & accept_vs[b]
            cvals = [cx1s[b], cy1s[b], cx2s[b], cy2s[b], cars[b]]
            for f in range(5):
                sel_refs[5 * b + f][...] = jnp.where(amask, cvals[f],
                                                     sxs[b][f])
            row = jnp.where(lane1 == 0, cbx1s[b], 0.0)
            row = jnp.where(lane1 == 1, cby1s[b], row)
            row = jnp.where(lane1 == 2, cbx2s[b], row)
            row = jnp.where(lane1 == 3, cby2s[b], row)
            row = jnp.where(lane1 == 4, ms[b], row)
            row = jnp.where(lane1 == 5, cclss[b], row)
            t_eff = jnp.where(accept_ss[b], t, _TOP_K)
            out_ref[b, pl.ds(t_eff, 1), :] = row
            new_ts.append(jnp.where(alive_ss[b],
                                    t + accept_ss[b].astype(jnp.int32),
                                    _TOP_K))
        return (tuple(new_ts), tuple(new_rms))

    jax.lax.while_loop(
        cond, body,
        (tuple(jnp.int32(0) for _ in range(B)), tuple(rm_init)))


@jax.jit
def kernel(confidences, regressions, anchors):
    B, N, C = confidences.shape
    R = (N + _LANES - 1) // _LANES
    NP = R * _LANES
    pad = NP - N

    blk = 1024
    while N % blk != 0 or blk % 8 != 0:
        blk -= 8
    sc, cl = pl.pallas_call(
        _score_cls_kernel,
        grid=(B, N // blk),
        in_specs=[pl.BlockSpec((1, blk, C), lambda b, i: (b, i, 0))],
        out_specs=[pl.BlockSpec((1, blk, 1), lambda b, i: (b, i, 0)),
                   pl.BlockSpec((1, blk, 1), lambda b, i: (b, i, 0))],
        out_shape=[jax.ShapeDtypeStruct((B, N, 1), jnp.float32),
                   jax.ShapeDtypeStruct((B, N, 1), jnp.float32)],
    )(confidences)
    scores = jnp.pad(sc[:, :, 0], ((0, 0), (0, pad))).reshape(B, R, _LANES)
    clsf = jnp.pad(cl[:, :, 0], ((0, 0), (0, pad))).reshape(B, R, _LANES)
    reg_p = jnp.pad(regressions, ((0, 0), (0, pad), (0, 0)))
    anch_p = jnp.pad(anchors, ((0, pad), (0, 0)))
    reg_t = reg_p.transpose(0, 2, 1).reshape(B, 4, R, _LANES)
    anch_t = anch_p.T.reshape(4, R, _LANES)

    (out_p,) = pl.pallas_call(
        _nms_kernel,
        grid=(1,),
        in_specs=[
            pl.BlockSpec((B, R, _LANES), lambda i: (0, 0, 0)),
            pl.BlockSpec((B, R, _LANES), lambda i: (0, 0, 0)),
            pl.BlockSpec((B, 4, R, _LANES), lambda i: (0, 0, 0, 0)),
            pl.BlockSpec((4, R, _LANES), lambda i: (0, 0, 0)),
        ],
        out_specs=[pl.BlockSpec((B, _OUTR, _LANES), lambda i: (0, 0, 0))],
        out_shape=[jax.ShapeDtypeStruct((B, _OUTR, _LANES), jnp.float32)],
        scratch_shapes=[pltpu.VMEM((B, R, _LANES), jnp.float32)] * 4
        + [pltpu.VMEM((R, _LANES), jnp.float32)] * B
        + [pltpu.VMEM((1, _SEL), jnp.float32)] * (5 * B),
    )(scores, clsf, reg_t, anch_t)
    out = out_p[:, :_TOP_K, 0:5]
    out_classes = out_p[:, :_TOP_K, 5].astype(jnp.int32)
    return out, out_classes


# speedup vs baseline: 5.1428x; 1.2122x over previous
import jax
import jax.numpy as jnp
from jax.experimental import pallas as pl
from jax.experimental.pallas import tpu as pltpu

_NUM_CLASSES = 90
_TOP_K = 200
_CONF = 0.05
_NMS_T = 0.5
_CROP = 300.0
_NEG = -1e9
_LANES = 128
_SEL = 256
_OUTR = 208


def _score_cls_kernel(conf_ref, score_ref, cls_ref):
    c = conf_ref[0]
    m = jnp.max(c, axis=1, keepdims=True)
    iota = jax.lax.broadcasted_iota(jnp.int32, c.shape, 1)
    idx = jnp.min(jnp.where(c == m, iota, 2147483647), axis=1, keepdims=True)
    score_ref[0] = m
    cls_ref[0] = idx.astype(jnp.float32)


def _nms_kernel(score_ref, cls_ref, reg_ref, anch_ref, out_ref, *scratch):
    B = score_ref.shape[0]
    R = score_ref.shape[1]
    bx1_ref, by1_ref, bx2_ref, by2_ref = scratch[0:4]
    s_refs = scratch[4:4 + B]
    sel_refs = scratch[4 + B:]

    scores_raw = score_ref[...]
    clsf = cls_ref[...]
    ay1 = anch_ref[0]
    ax1 = anch_ref[1]
    ay2 = anch_ref[2]
    ax2 = anch_ref[3]
    dy = reg_ref[:, 0]
    dx = reg_ref[:, 1]
    dh = reg_ref[:, 2]
    dw = reg_ref[:, 3]
    yc_a = (ay1 + ay2) / 2.0
    xc_a = (ax1 + ax2) / 2.0
    ha = ay2 - ay1
    wa = ax2 - ax1
    w = jnp.exp(dw) * wa
    h = jnp.exp(dh) * ha
    yc = dy * ha + yc_a
    xc = dx * wa + xc_a
    bx1_ref[...] = jnp.clip(xc - w / 2.0, 0.0, _CROP) / _CROP
    by1_ref[...] = jnp.clip(yc - h / 2.0, 0.0, _CROP) / _CROP
    bx2_ref[...] = jnp.clip(xc + w / 2.0, 0.0, _CROP) / _CROP
    by2_ref[...] = jnp.clip(yc + h / 2.0, 0.0, _CROP) / _CROP

    lane1 = jax.lax.broadcasted_iota(jnp.int32, (1, _LANES), 1)
    iota_sel = jax.lax.broadcasted_iota(jnp.int32, (1, _SEL), 1)
    s0 = jnp.where(scores_raw > _CONF, scores_raw, _NEG)
    zsel = jnp.zeros((1, _SEL), jnp.float32)
    for b in range(B):
        s_refs[b][...] = s0[b]
        for f in range(5):
            sel_refs[5 * b + f][...] = zsel
    out_ref[...] = jnp.broadcast_to(
        jnp.where(lane1 == 5, -1.0, 0.0)[None], (B, _OUTR, _LANES))

    rm0 = jnp.max(s0, axis=2)
    rm_init = [
        jnp.concatenate(
            [rm0[b:b + 1, :], jnp.full((1, _SEL - R), _NEG, jnp.float32)],
            axis=1)
        for b in range(B)
    ]

    def rowsel(ref3, b, ri, lmask):
        row = ref3[b, pl.ds(ri, 1), :]
        return jnp.sum(jnp.where(lmask, row, 0.0))

    def cond(state):
        ts, _ = state
        go = False
        for b in range(B):
            go = go | (ts[b] < _TOP_K)
        return go

    def body(state):
        ts, rms = state
        bs = range(B)
        ms = [jnp.max(rms[b]) for b in bs]
        alive_vs = [ms[b] > _NEG / 2.0 for b in bs]
        rid_vs = [jnp.min(jnp.where(rms[b] == ms[b], iota_sel, 2147483647))
                  for b in bs]
        rids = [jax.lax.convert_element_type(rid_vs[b], jnp.int32) for b in bs]
        srows = [s_refs[b][pl.ds(rids[b], 1), :] for b in bs]
        li_vs = [jnp.min(jnp.where(srows[b] == ms[b], lane1, 2147483647))
                 for b in bs]
        lmasks = [lane1 == li_vs[b] for b in bs]
        nrows = [jnp.where(lmasks[b], _NEG, srows[b]) for b in bs]
        for b in bs:
            s_refs[b][pl.ds(rids[b], 1), :] = nrows[b]
        nrmaxs = [jnp.max(nrows[b]) for b in bs]
        new_rms = [jnp.where(iota_sel == rid_vs[b], nrmaxs[b], rms[b])
                   for b in bs]
        cbx1s = [rowsel(bx1_ref, b, rids[b], lmasks[b]) for b in bs]
        cby1s = [rowsel(by1_ref, b, rids[b], lmasks[b]) for b in bs]
        cbx2s = [rowsel(bx2_ref, b, rids[b], lmasks[b]) for b in bs]
        cby2s = [rowsel(by2_ref, b, rids[b], lmasks[b]) for b in bs]
        cclss = [rowsel(cls_ref, b, rids[b], lmasks[b]) for b in bs]
        coffs = [cclss[b] * 2.0 for b in bs]
        cx1s = [cbx1s[b] + coffs[b] for b in bs]
        cy1s = [cby1s[b] + coffs[b] for b in bs]
        cx2s = [cbx2s[b] + coffs[b] for b in bs]
        cy2s = [cby2s[b] + coffs[b] for b in bs]
        cars = [(jnp.clip(cx2s[b] - cx1s[b], 0.0, None)
                 * jnp.clip(cy2s[b] - cy1s[b], 0.0, None)) for b in bs]
        sxs = [[sel_refs[5 * b + f][...] for f in range(5)] for b in bs]
        sups = []
        for b in bs:
            sx1, sy1, sx2, sy2, sar = sxs[b]
            xx1 = jnp.maximum(cx1s[b], sx1)
            yy1 = jnp.maximum(cy1s[b], sy1)
            xx2 = jnp.minimum(cx2s[b], sx2)
            yy2 = jnp.minimum(cy2s[b], sy2)
            inter = (jnp.clip(xx2 - xx1, 0.0, None)
                     * jnp.clip(yy2 - yy1, 0.0, None))
            iou = inter / (sar + cars[b] - inter + 1e-8)
            sups.append(jnp.max(jnp.where(iou > _NMS_T, 1, 0)) > 0)
        accept_vs = [alive_vs[b] & jnp.logical_not(sups[b]) & (ts[b] < _TOP_K)
                     for b in bs]
        flagss = [(jnp.where(accept_vs[b], 2, 0)
                   + jnp.where(alive_vs[b], 1, 0)) for b in bs]
        fss = [jax.lax.convert_element_type(flagss[b], jnp.int32) for b in bs]
        accept_ss = [fss[b] == 3 for b in bs]
        alive_ss = [fss[b] >= 1 for b in bs]
        new_ts = []
        for b in bs:
            t = ts[b]
            amask = (iota_sel == t) & accept_vs[b]
            cvals = [cx1s[b], cy1s[b], cx2s[b], cy2s[b], cars[b]]
            for f in range(5):
                sel_refs[5 * b + f][...] = jnp.where(amask, cvals[f],
                                                     sxs[b][f])
            row = jnp.where(lane1 == 0, cbx1s[b], 0.0)
            row = jnp.where(lane1 == 1, cby1s[b], row)
            row = jnp.where(lane1 == 2, cbx2s[b], row)
            row = jnp.where(lane1 == 3, cby2s[b], row)
            row = jnp.where(lane1 == 4, ms[b], row)
            row = jnp.where(lane1 == 5, cclss[b], row)
            t_eff = jnp.where(accept_ss[b], t, _TOP_K)
            out_ref[b, pl.ds(t_eff, 1), :] = row
            new_ts.append(jnp.where(alive_ss[b],
                                    t + accept_ss[b].astype(jnp.int32),
                                    _TOP_K))
        return (tuple(new_ts), tuple(new_rms))

    jax.lax.while_loop(
        cond, body,
        (tuple(jnp.int32(0) for _ in range(B)), tuple(rm_init)))


@jax.jit
def kernel(confidences, regressions, anchors):
    B, N, C = confidences.shape
    R = (N + _LANES - 1) // _LANES
    NP = R * _LANES
    pad = NP - N

    blk = 10000
    while N % blk != 0 or blk % 8 != 0:
        blk -= 8
    sc, cl = pl.pallas_call(
        _score_cls_kernel,
        grid=(B, N // blk),
        in_specs=[pl.BlockSpec((1, blk, C), lambda b, i: (b, i, 0))],
        out_specs=[pl.BlockSpec((1, blk, 1), lambda b, i: (b, i, 0)),
                   pl.BlockSpec((1, blk, 1), lambda b, i: (b, i, 0))],
        out_shape=[jax.ShapeDtypeStruct((B, N, 1), jnp.float32),
                   jax.ShapeDtypeStruct((B, N, 1), jnp.float32)],
    )(confidences)
    scores = jnp.pad(sc[:, :, 0], ((0, 0), (0, pad))).reshape(B, R, _LANES)
    clsf = jnp.pad(cl[:, :, 0], ((0, 0), (0, pad))).reshape(B, R, _LANES)
    reg_p = jnp.pad(regressions, ((0, 0), (0, pad), (0, 0)))
    anch_p = jnp.pad(anchors, ((0, pad), (0, 0)))
    reg_t = reg_p.transpose(0, 2, 1).reshape(B, 4, R, _LANES)
    anch_t = anch_p.T.reshape(4, R, _LANES)

    (out_p,) = pl.pallas_call(
        _nms_kernel,
        grid=(1,),
        in_specs=[
            pl.BlockSpec((B, R, _LANES), lambda i: (0, 0, 0)),
            pl.BlockSpec((B, R, _LANES), lambda i: (0, 0, 0)),
            pl.BlockSpec((B, 4, R, _LANES), lambda i: (0, 0, 0, 0)),
            pl.BlockSpec((4, R, _LANES), lambda i: (0, 0, 0)),
        ],
        out_specs=[pl.BlockSpec((B, _OUTR, _LANES), lambda i: (0, 0, 0))],
        out_shape=[jax.ShapeDtypeStruct((B, _OUTR, _LANES), jnp.float32)],
        scratch_shapes=[pltpu.VMEM((B, R, _LANES), jnp.float32)] * 4
        + [pltpu.VMEM((R, _LANES), jnp.float32)] * B
        + [pltpu.VMEM((1, _SEL), jnp.float32)] * (5 * B),
    )(scores, clsf, reg_t, anch_t)
    out = out_p[:, :_TOP_K, 0:5]
    out_classes = out_p[:, :_TOP_K, 5].astype(jnp.int32)
    return out, out_classes
